# baseline (speedup 1.0000x reference)
import jax
import jax.numpy as jnp
from jax.experimental import pallas as pl
from jax.experimental.pallas import tpu as pltpu

_TARGET_RATE = 0.75
_TEMPERATURE = 0.5
_ONE_F32_BITS = 0x3F800000


def _body(tr_ref, imp_ref, u_ref, sp_ref, mm_ref, os_ref, om_ref, omask_ref,
          *, k, B, HW):
    i = pl.program_id(0)

    @pl.when(i == 0)
    def _compute_mask():
        impv = imp_ref[...]
        bits = jax.lax.bitcast_convert_type(impv, jnp.int32)

        lo = jnp.zeros((B, 1), jnp.int32)
        hi = jnp.full((B, 1), _ONE_F32_BITS, jnp.int32)

        def vstep(_, c):
            vlo, vhi = c
            mid = (vlo + vhi) >> 1
            cnt = jnp.sum((bits >= mid).astype(jnp.int32), axis=1, keepdims=True)
            ge = cnt >= k
            return jnp.where(ge, mid, vlo), jnp.where(ge, vhi, mid)

        lo, hi = jax.lax.fori_loop(0, 30, vstep, (lo, hi))
        m = lo
        gt = bits > m
        eq = bits == m
        c_gt = jnp.sum(gt.astype(jnp.int32), axis=1, keepdims=True)
        extra = k - c_gt

        idx = jax.lax.broadcasted_iota(jnp.int32, (B, HW), 1)
        lo2 = jnp.full((B, 1), -1, jnp.int32)
        hi2 = jnp.full((B, 1), HW - 1, jnp.int32)

        def istep(_, c):
            ilo, ihi = c
            mid = (ilo + ihi) >> 1
            cnt = jnp.sum((eq & (idx <= mid)).astype(jnp.int32), axis=1,
                          keepdims=True)
            ge = cnt >= extra
            return jnp.where(ge, ilo, mid), jnp.where(ge, mid, ihi)

        lo2, hi2 = jax.lax.fori_loop(0, 12, istep, (lo2, hi2))
        mask_eval = (gt | (eq & (idx <= hi2))).astype(jnp.float32)

        uv = u_ref[...]
        logits = jnp.log(impv / (1.0 - impv + 1e-07) + 1e-07)
        soft = jax.nn.sigmoid((logits - jnp.log(-jnp.log(uv))) / _TEMPERATURE)
        mask_train = (soft > 0.5).astype(jnp.float32)

        omask_ref[...] = jnp.where(tr_ref[0] != 0, mask_train, mask_eval)

    b = jax.lax.rem(i, B)
    mall = omask_ref[...]
    rows = jax.lax.broadcasted_iota(jnp.int32, (B, HW), 0)
    row = jnp.sum(jnp.where(rows == b, mall, 0.0), axis=0, keepdims=True)
    row3 = row[:, None, :]
    os_ref[...] = sp_ref[...] * row3
    om_ref[...] = mm_ref[...] * row3


def kernel(spikes, mems, importance, training=0, target_override=None):
    target = target_override if target_override is not None else _TARGET_RATE
    T, B, C, H, W = spikes.shape
    HW = H * W
    k = max(1, int(target * HW))

    sp = spikes.reshape(T * B, C, HW)
    mm = mems.reshape(T * B, C, HW)
    imp = importance.reshape(B, HW)
    u = jnp.clip(
        jax.random.uniform(jax.random.key(42), (B, 1, H, W), dtype=jnp.float32),
        1e-07, 1.0 - 1e-07).reshape(B, HW)
    tr = jnp.asarray(training, jnp.int32).reshape(1)

    import functools
    body = functools.partial(_body, k=k, B=B, HW=HW)

    os_, om_, omask = pl.pallas_call(
        body,
        grid=(T * B,),
        in_specs=[
            pl.BlockSpec(memory_space=pltpu.SMEM),
            pl.BlockSpec((B, HW), lambda i: (0, 0)),
            pl.BlockSpec((B, HW), lambda i: (0, 0)),
            pl.BlockSpec((1, C, HW), lambda i: (i, 0, 0)),
            pl.BlockSpec((1, C, HW), lambda i: (i, 0, 0)),
        ],
        out_specs=[
            pl.BlockSpec((1, C, HW), lambda i: (i, 0, 0)),
            pl.BlockSpec((1, C, HW), lambda i: (i, 0, 0)),
            pl.BlockSpec((B, HW), lambda i: (0, 0)),
        ],
        out_shape=[
            jax.ShapeDtypeStruct((T * B, C, HW), jnp.float32),
            jax.ShapeDtypeStruct((T * B, C, HW), jnp.float32),
            jax.ShapeDtypeStruct((B, HW), jnp.float32),
        ],
    )(tr, imp, u, sp, mm)

    return (os_.reshape(T, B, C, H, W), om_.reshape(T, B, C, H, W),
            omask.reshape(B, 1, H, W))


# speedup vs baseline: 1.0249x; 1.0249x over previous
import jax
import jax.numpy as jnp
from jax.experimental import pallas as pl
from jax.experimental.pallas import tpu as pltpu

_TARGET_RATE = 0.75
_TEMPERATURE = 0.5
_ONE_F32_BITS = 0x3F800000


def _body(tr_ref, imp_ref, u_ref, sp_ref, mm_ref, os_ref, om_ref, omask_ref,
          *, k, B, HW):
    i = pl.program_id(0)
    j = pl.program_id(1)

    @pl.when((i == 0) & (j == 0))
    def _compute_mask():
        impv = imp_ref[...]
        bits = jax.lax.bitcast_convert_type(impv, jnp.int32)

        lo = jnp.zeros((B, 1), jnp.int32)
        hi = jnp.full((B, 1), _ONE_F32_BITS, jnp.int32)

        def vstep(_, c):
            vlo, vhi = c
            mid = (vlo + vhi) >> 1
            cnt = jnp.sum((bits >= mid).astype(jnp.int32), axis=1, keepdims=True)
            ge = cnt >= k
            return jnp.where(ge, mid, vlo), jnp.where(ge, vhi, mid)

        lo, hi = jax.lax.fori_loop(0, 30, vstep, (lo, hi))
        m = lo
        gt = bits > m
        eq = bits == m
        c_gt = jnp.sum(gt.astype(jnp.int32), axis=1, keepdims=True)
        extra = k - c_gt

        idx = jax.lax.broadcasted_iota(jnp.int32, (B, HW), 1)
        lo2 = jnp.full((B, 1), -1, jnp.int32)
        hi2 = jnp.full((B, 1), HW - 1, jnp.int32)

        def istep(_, c):
            ilo, ihi = c
            mid = (ilo + ihi) >> 1
            cnt = jnp.sum((eq & (idx <= mid)).astype(jnp.int32), axis=1,
                          keepdims=True)
            ge = cnt >= extra
            return jnp.where(ge, ilo, mid), jnp.where(ge, mid, ihi)

        lo2, hi2 = jax.lax.fori_loop(0, 12, istep, (lo2, hi2))
        mask_eval = (gt | (eq & (idx <= hi2))).astype(jnp.float32)

        uv = u_ref[...]
        logits = jnp.log(impv / (1.0 - impv + 1e-07) + 1e-07)
        soft = jax.nn.sigmoid((logits - jnp.log(-jnp.log(uv))) / _TEMPERATURE)
        mask_train = (soft > 0.5).astype(jnp.float32)

        omask_ref[...] = jnp.where(tr_ref[0] != 0, mask_train, mask_eval)

    mall = omask_ref[...][None, :, None, :]
    os_ref[...] = sp_ref[...] * mall
    om_ref[...] = mm_ref[...] * mall


def kernel(spikes, mems, importance, training=0, target_override=None):
    target = target_override if target_override is not None else _TARGET_RATE
    T, B, C, H, W = spikes.shape
    HW = H * W
    k = max(1, int(target * HW))

    CH = C // 2
    sp = spikes.reshape(T, B, C, HW)
    mm = mems.reshape(T, B, C, HW)
    imp = importance.reshape(B, HW)
    u = jnp.clip(
        jax.random.uniform(jax.random.key(42), (B, 1, H, W), dtype=jnp.float32),
        1e-07, 1.0 - 1e-07).reshape(B, HW)
    tr = jnp.asarray(training, jnp.int32).reshape(1)

    import functools
    body = functools.partial(_body, k=k, B=B, HW=HW)

    os_, om_, omask = pl.pallas_call(
        body,
        grid=(T, C // CH),
        in_specs=[
            pl.BlockSpec(memory_space=pltpu.SMEM),
            pl.BlockSpec((B, HW), lambda i, j: (0, 0)),
            pl.BlockSpec((B, HW), lambda i, j: (0, 0)),
            pl.BlockSpec((1, B, CH, HW), lambda i, j: (i, 0, j, 0)),
            pl.BlockSpec((1, B, CH, HW), lambda i, j: (i, 0, j, 0)),
        ],
        out_specs=[
            pl.BlockSpec((1, B, CH, HW), lambda i, j: (i, 0, j, 0)),
            pl.BlockSpec((1, B, CH, HW), lambda i, j: (i, 0, j, 0)),
            pl.BlockSpec((B, HW), lambda i, j: (0, 0)),
        ],
        out_shape=[
            jax.ShapeDtypeStruct((T, B, C, HW), jnp.float32),
            jax.ShapeDtypeStruct((T, B, C, HW), jnp.float32),
            jax.ShapeDtypeStruct((B, HW), jnp.float32),
        ],
    )(tr, imp, u, sp, mm)

    return (os_.reshape(T, B, C, H, W), om_.reshape(T, B, C, H, W),
            omask.reshape(B, 1, H, W))
